# baseline (speedup 1.0000x reference)
import functools

import numpy as np

import jax
import jax.numpy as jnp
from jax import lax
from jax.experimental import pallas as pl
from jax.experimental.pallas import tpu as pltpu
from jax.experimental.pallas import tpu_sc as plsc

_DIM = 2048
_HEADS = 8
_NK = 128
_TK = 16
_DH = _DIM // _HEADS

_TBLK = 512


def _proj_score_body(x_ref, wq_ref, k_ref, s_ref, i_ref):
    xb = x_ref[0]
    wq = wq_ref[0]
    kk = k_ref[0]
    q = lax.dot_general(xb, wq, (((1,), (1,)), ((), ())),
                        preferred_element_type=jnp.float32)
    d_t = lax.dot_general(kk, q, (((1,), (1,)), ((), ())),
                          preferred_element_type=jnp.float32)
    work = d_t
    iot = lax.broadcasted_iota(jnp.int32, (_NK, _TBLK), 0)
    svals, sidx = [], []
    for _k in range(_TK):
        m = jnp.max(work, axis=0)
        eq = work == m[None, :]
        mi = jnp.min(jnp.where(eq, iot, _NK), axis=0)
        svals.append(m)
        sidx.append(mi)
        work = jnp.where(iot == mi[None, :], jnp.float32(-3.0e38), work)
    s_ref[0, 0] = jnp.stack(svals, axis=0).T
    i_ref[0, 0] = jnp.stack(sidx, axis=0).T


def _proj_score(x, wq_r, k_r):
    b, t, e = x.shape
    grid = (b, t // _TBLK, 2 * _HEADS)
    return pl.pallas_call(
        _proj_score_body,
        grid=grid,
        in_specs=[
            pl.BlockSpec((1, _TBLK, e), lambda ib, it, iph: (ib, it, 0)),
            pl.BlockSpec((1, _DH, e), lambda ib, it, iph: (iph, 0, 0)),
            pl.BlockSpec((1, _NK, _DH), lambda ib, it, iph: (iph, 0, 0)),
        ],
        out_specs=[
            pl.BlockSpec((1, 1, _TBLK, _TK), lambda ib, it, iph: (ib, iph, it, 0)),
            pl.BlockSpec((1, 1, _TBLK, _TK), lambda ib, it, iph: (ib, iph, it, 0)),
        ],
        out_shape=[
            jax.ShapeDtypeStruct((b, 2 * _HEADS, t, _TK), jnp.float32),
            jax.ShapeDtypeStruct((b, 2 * _HEADS, t, _TK), jnp.int32),
        ],
    )(x, wq_r, k_r)


_TBLK2 = 512


def _out_proj_body(xh_ref, we_ref, wo_ref, bo_ref, y_ref):
    ih = pl.program_id(2)
    gi = lax.bitcast_convert_type(xh_ref[0, 0], jnp.int32)
    ge = lax.bitcast_convert_type(gi << 16, jnp.float32)
    go = lax.bitcast_convert_type(gi & jnp.int32(-65536), jnp.float32)
    part = lax.dot_general(ge, we_ref[...], (((1,), (1,)), ((), ())),
                           preferred_element_type=jnp.float32)
    part += lax.dot_general(go, wo_ref[...], (((1,), (1,)), ((), ())),
                            preferred_element_type=jnp.float32)

    @pl.when(ih == 0)
    def _():
        y_ref[0] = part + bo_ref[0]

    @pl.when(ih != 0)
    def _():
        y_ref[0] += part


def _out_proj(out_words, we_r, wo_r, bo_r):
    b = out_words.shape[0]
    t = out_words.shape[2]
    grid = (b, t // _TBLK2, _HEADS)
    hw = _DH // 2
    return pl.pallas_call(
        _out_proj_body,
        grid=grid,
        in_specs=[
            pl.BlockSpec((1, 1, _TBLK2, hw), lambda ib, it, ih: (ib, ih, it, 0)),
            pl.BlockSpec((_DIM, hw), lambda ib, it, ih: (0, ih)),
            pl.BlockSpec((_DIM, hw), lambda ib, it, ih: (0, ih)),
            pl.BlockSpec((1, _DIM), lambda ib, it, ih: (0, 0)),
        ],
        out_specs=pl.BlockSpec((1, _TBLK2, _DIM), lambda ib, it, ih: (ib, it, 0)),
        out_shape=jax.ShapeDtypeStruct((b, t, _DIM), jnp.float32),
    )(out_words, we_r, wo_r, bo_r)


_CT = 16
_NCH = 512 // _CT

def _sortd(k, v):
    return plsc.sort_key_val(k, v, descending=True)


def _tophalf(tv, ti, cv, ci):
    rcv = jnp.flip(cv, 0)
    rci = jnp.flip(ci, 0)
    m = tv >= rcv
    return jnp.where(m, tv, rcv), jnp.where(m, ti, rci)


def _merge16(tv, ti, cv, ci):
    nv, ni = _tophalf(tv, ti, cv, ci)
    return _sortd(nv, ni)


def _combine(tv0, ti0, tv1, ti1):
    rows = [(tv1 + tv0[i], ti1 + ti0[i] * _NK) for i in range(4)]
    colmask = lax.iota(jnp.int32, 16) >= 4
    cols = []
    for j in range(3):
        cv = tv0 + tv1[j]
        ci = ti0 * _NK + ti1[j]
        sv, si, om = plsc.sort_key_val(cv, ci, mask=colmask, descending=True)
        cols.append((jnp.where(om, sv, jnp.float32(-3.0e38)), si))
    mr = _merge16(*_merge16(*rows[0], *rows[1]), *_merge16(*rows[2], *rows[3]))
    mc = _merge16(*_merge16(*cols[0], *cols[1]), *cols[2])
    return _tophalf(*mr, *mc)


def _pkm_sc(s_flat, i_flat, values_flat, n_rows_out):
    mesh = plsc.VectorSubcoreMesh(core_axis_name="c", subcore_axis_name="s",
                                  num_cores=2, num_subcores=16)

    @functools.partial(
        pl.kernel,
        out_type=jax.ShapeDtypeStruct((n_rows_out, _DH // 2), jnp.float32),
        mesh=mesh,
        compiler_params=pltpu.CompilerParams(needs_layout_passes=False),
        scratch_types=[
            pltpu.VMEM((2, 4, _CT, _TK), jnp.float32),
            pltpu.VMEM((2, 4, _CT, _TK), jnp.int32),
            pltpu.VMEM((2 * _CT, _DH // 2), jnp.float32),
            pltpu.VMEM((2, 32, _DH // 2), jnp.float32),
            pltpu.VMEM((2, 32), jnp.int32),
            pltpu.SemaphoreType.DMA,
            pltpu.SemaphoreType.DMA,
            pltpu.SemaphoreType.DMA,
        ],
    )
    def body(s_hbm, i_hbm, values_hbm, out_hbm,
             ssv, ssi, outb, gb, vib, sem_in, sem_g0, sem_g1):
        c = lax.axis_index("c")
        s = lax.axis_index("s")
        b = c
        h = s // 2
        half = s % 2
        row_p = [(b * 16 + h) * 2048, (b * 16 + 8 + h) * 2048]
        tau0 = half * 512
        out_base = (b * 8 + h) * 2048
        vbase = h * (_NK * _NK)
        sem_g = [sem_g0, sem_g1]

        def issue_stage(ci_, par):
            t0 = tau0 + ci_ * _CT
            for p in range(2):
                for seg in range(2):
                    rb = row_p[p] + seg * 1024 + t0
                    pltpu.async_copy(s_hbm.at[pl.ds(rb, _CT), :],
                                     ssv.at[par, p * 2 + seg], sem_in)
                    pltpu.async_copy(i_hbm.at[pl.ds(rb, _CT), :],
                                     ssi.at[par, p * 2 + seg], sem_in)

        issue_stage(0, 0)

        @pl.loop(0, _NCH)
        def _chunk(ci_):
            par = lax.rem(ci_, 2)
            t0 = tau0 + ci_ * _CT
            for k in range(4):
                pltpu.make_async_copy(s_hbm.at[pl.ds(0, _CT), :],
                                      ssv.at[par, k], sem_in).wait()
                pltpu.make_async_copy(i_hbm.at[pl.ds(0, _CT), :],
                                      ssi.at[par, k], sem_in).wait()

            @pl.when(ci_ + 1 < _NCH)
            def _():
                issue_stage(ci_ + 1, 1 - par)

            @pl.loop(0, _CT // 2)
            def _pair(tp):
                rows = ((2 * tp, 0), (2 * tp + 1, 1))
                atts = [[None, None], [None, None]]
                for tl, bufi in rows:
                    for p in range(2):
                        tv0 = ssv[par, p * 2 + 0, tl, :]
                        ti0 = ssi[par, p * 2 + 0, tl, :]
                        tv1 = ssv[par, p * 2 + 1, tl, :]
                        ti1 = ssi[par, p * 2 + 1, tl, :]
                        rv, ri = _combine(tv0, ti0, tv1, ti1)
                        mx = jnp.max(rv)
                        ev = jnp.exp(rv - mx)
                        atts[bufi][p] = ev / jnp.sum(ev)
                        vib[bufi, pl.ds(p * 16, 16)] = ri + vbase
                    pltpu.async_copy(values_hbm.at[vib.at[bufi]],
                                     gb.at[bufi], sem_g[bufi])
                for tl, bufi in rows:
                    pltpu.make_async_copy(values_hbm.at[vib.at[bufi]],
                                          gb.at[bufi], sem_g[bufi]).wait()
                    maskhi = jnp.int32(-65536)
                    half = jnp.int32(0x8000)
                    for p in range(2):
                        at = atts[bufi][p]
                        acca = [None] * 8
                        accb = [None] * 8
                        for j in range(16):
                            aj = at[j]
                            for dc in range(8):
                                gw = gb[bufi, p * 16 + j, pl.ds(dc * 16, 16)]
                                gi = plsc.bitcast(gw, jnp.int32)
                                ga = plsc.bitcast(gi << 16, jnp.float32)
                                gc = plsc.bitcast(gi & maskhi, jnp.float32)
                                ta = aj * ga
                                tb = aj * gc
                                acca[dc] = ta if j == 0 else acca[dc] + ta
                                accb[dc] = tb if j == 0 else accb[dc] + tb
                        for dc in range(8):
                            ia = plsc.bitcast(acca[dc], jnp.int32)
                            ib2 = plsc.bitcast(accb[dc], jnp.int32)
                            lo = lax.shift_right_logical(ia + half, 16)
                            hi = (ib2 + half) & maskhi
                            outb[2 * tl + p, pl.ds(dc * 16, 16)] = plsc.bitcast(
                                hi | lo, jnp.float32)

            pltpu.sync_copy(outb, out_hbm.at[pl.ds(out_base + 2 * t0, 2 * _CT), :])

    return body(s_flat, i_flat, values_flat)


def kernel(x, W_q, W_o, b_o, keys, values):
    b, t, e = x.shape
    wq_r = W_q.reshape(2 * _HEADS, _DH, e)
    k_r = jnp.transpose(keys, (2, 0, 1, 3)).reshape(2 * _HEADS, _NK, _DH)
    sc_top, idx_top = _proj_score(x, wq_r, k_r)
    s_flat = sc_top.reshape(b * 2 * _HEADS * t, _TK)
    i_flat = idx_top.reshape(b * 2 * _HEADS * t, _TK)
    vi = lax.bitcast_convert_type(values.reshape(_HEADS * _NK * _NK, _DH),
                                  jnp.int32) + jnp.int32(0x8000)
    vw = (vi[:, 1::2] & jnp.int32(-65536)) | lax.shift_right_logical(
        vi[:, 0::2], 16)
    values_flat = lax.bitcast_convert_type(vw, jnp.float32)
    out_words = _pkm_sc(s_flat, i_flat, values_flat,
                        b * _HEADS * t)
    out_words = out_words.reshape(b, _HEADS, t, _DH // 2)
    y = _out_proj(out_words, W_o[:, 0::2], W_o[:, 1::2], b_o.reshape(1, _DIM))
    return y


# speedup vs baseline: 5.4007x; 5.4007x over previous
import functools

import numpy as np

import jax
import jax.numpy as jnp
from jax import lax
from jax.experimental import pallas as pl
from jax.experimental.pallas import tpu as pltpu
from jax.experimental.pallas import tpu_sc as plsc

_DIM = 2048
_HEADS = 8
_NK = 128
_TK = 16
_DH = _DIM // _HEADS

_TBLK = 512


def _proj_score_body(x_ref, wq_ref, k_ref, s_ref, i_ref):
    xb = x_ref[0]
    wq = wq_ref[0]
    kk = k_ref[0]
    q = lax.dot_general(xb, wq, (((1,), (1,)), ((), ())),
                        preferred_element_type=jnp.float32)
    d_t = lax.dot_general(kk, q, (((1,), (1,)), ((), ())),
                          preferred_element_type=jnp.float32)
    work = d_t
    iot = lax.broadcasted_iota(jnp.int32, (_NK, _TBLK), 0)
    svals, sidx = [], []
    for _k in range(_TK):
        m = jnp.max(work, axis=0)
        eq = work == m[None, :]
        mi = jnp.min(jnp.where(eq, iot, _NK), axis=0)
        svals.append(m)
        sidx.append(mi)
        work = jnp.where(iot == mi[None, :], jnp.float32(-3.0e38), work)
    s_ref[0, 0] = jnp.stack(svals, axis=0).T
    i_ref[0, 0] = jnp.stack(sidx, axis=0).T


def _proj_score(x, wq_r, k_r):
    b, t, e = x.shape
    grid = (b, t // _TBLK, 2 * _HEADS)
    return pl.pallas_call(
        _proj_score_body,
        grid=grid,
        in_specs=[
            pl.BlockSpec((1, _TBLK, e), lambda ib, it, iph: (ib, it, 0)),
            pl.BlockSpec((1, _DH, e), lambda ib, it, iph: (iph, 0, 0)),
            pl.BlockSpec((1, _NK, _DH), lambda ib, it, iph: (iph, 0, 0)),
        ],
        out_specs=[
            pl.BlockSpec((1, 1, _TBLK, _TK), lambda ib, it, iph: (ib, iph, it, 0)),
            pl.BlockSpec((1, 1, _TBLK, _TK), lambda ib, it, iph: (ib, iph, it, 0)),
        ],
        out_shape=[
            jax.ShapeDtypeStruct((b, 2 * _HEADS, t, _TK), jnp.float32),
            jax.ShapeDtypeStruct((b, 2 * _HEADS, t, _TK), jnp.int32),
        ],
    )(x, wq_r, k_r)


_TBLK2 = 512


def _out_proj_body(xh_ref, we_ref, wo_ref, bo_ref, y_ref):
    ih = pl.program_id(2)
    gi = lax.bitcast_convert_type(xh_ref[0, 0], jnp.int32)
    ge = lax.bitcast_convert_type(gi << 16, jnp.float32)
    go = lax.bitcast_convert_type(gi & jnp.int32(-65536), jnp.float32)
    part = lax.dot_general(ge, we_ref[...], (((1,), (1,)), ((), ())),
                           preferred_element_type=jnp.float32)
    part += lax.dot_general(go, wo_ref[...], (((1,), (1,)), ((), ())),
                            preferred_element_type=jnp.float32)

    @pl.when(ih == 0)
    def _():
        y_ref[0] = part + bo_ref[0]

    @pl.when(ih != 0)
    def _():
        y_ref[0] += part


def _out_proj(out_words, wo, bo_r):
    b = out_words.shape[0]
    t = out_words.shape[2]
    grid = (b, t // _TBLK2, _HEADS)
    hw = _DH // 2
    return pl.pallas_call(
        _out_proj_body,
        grid=grid,
        in_specs=[
            pl.BlockSpec((1, 1, _TBLK2, hw), lambda ib, it, ih: (ib, ih, it, 0)),
            pl.BlockSpec((_DIM, hw), lambda ib, it, ih: (0, 2 * ih)),
            pl.BlockSpec((_DIM, hw), lambda ib, it, ih: (0, 2 * ih + 1)),
            pl.BlockSpec((1, _DIM), lambda ib, it, ih: (0, 0)),
        ],
        out_specs=pl.BlockSpec((1, _TBLK2, _DIM), lambda ib, it, ih: (ib, it, 0)),
        out_shape=jax.ShapeDtypeStruct((b, t, _DIM), jnp.float32),
    )(out_words, wo, wo, bo_r)


_CT = 16
_NCH = 512 // _CT

def _sortd(k, v):
    return plsc.sort_key_val(k, v, descending=True)


def _tophalf(tv, ti, cv, ci):
    rcv = jnp.flip(cv, 0)
    rci = jnp.flip(ci, 0)
    m = tv >= rcv
    return jnp.where(m, tv, rcv), jnp.where(m, ti, rci)


def _merge16(tv, ti, cv, ci):
    nv, ni = _tophalf(tv, ti, cv, ci)
    return _sortd(nv, ni)


def _combine(tv0, ti0, tv1, ti1):
    rows = [(tv1 + tv0[i], ti1 + ti0[i] * _NK) for i in range(4)]
    colmask = lax.iota(jnp.int32, 16) >= 4
    cols = []
    for j in range(3):
        cv = tv0 + tv1[j]
        ci = ti0 * _NK + ti1[j]
        sv, si, om = plsc.sort_key_val(cv, ci, mask=colmask, descending=True)
        cols.append((jnp.where(om, sv, jnp.float32(-3.0e38)), si))
    mr = _merge16(*_merge16(*rows[0], *rows[1]), *_merge16(*rows[2], *rows[3]))
    mc = _merge16(*_merge16(*cols[0], *cols[1]), *cols[2])
    return _tophalf(*mr, *mc)


def _pkm_sc(s_flat, i_flat, values_flat, n_rows_out):
    mesh = plsc.VectorSubcoreMesh(core_axis_name="c", subcore_axis_name="s",
                                  num_cores=2, num_subcores=16)

    @functools.partial(
        pl.kernel,
        out_type=jax.ShapeDtypeStruct((n_rows_out, _DH // 2), jnp.float32),
        mesh=mesh,
        compiler_params=pltpu.CompilerParams(needs_layout_passes=False),
        scratch_types=[
            pltpu.VMEM((2, 4, _CT, _TK), jnp.float32),
            pltpu.VMEM((2, 4, _CT, _TK), jnp.int32),
            pltpu.VMEM((2 * _CT, _DH // 2), jnp.float32),
            pltpu.VMEM((2, 32, _DH // 2), jnp.float32),
            pltpu.VMEM((2, 32), jnp.int32),
            pltpu.SemaphoreType.DMA,
            pltpu.SemaphoreType.DMA,
            pltpu.SemaphoreType.DMA,
        ],
    )
    def body(s_hbm, i_hbm, values_hbm, out_hbm,
             ssv, ssi, outb, gb, vib, sem_in, sem_g0, sem_g1):
        c = lax.axis_index("c")
        s = lax.axis_index("s")
        b = c
        h = s // 2
        half = s % 2
        row_p = [(b * 16 + h) * 2048, (b * 16 + 8 + h) * 2048]
        tau0 = half * 512
        out_base = (b * 8 + h) * 2048
        vbase = h * (_NK * _NK)
        sem_g = [sem_g0, sem_g1]

        def issue_stage(ci_, par):
            t0 = tau0 + ci_ * _CT
            for p in range(2):
                for seg in range(2):
                    rb = row_p[p] + seg * 1024 + t0
                    pltpu.async_copy(s_hbm.at[pl.ds(rb, _CT), :],
                                     ssv.at[par, p * 2 + seg], sem_in)
                    pltpu.async_copy(i_hbm.at[pl.ds(rb, _CT), :],
                                     ssi.at[par, p * 2 + seg], sem_in)

        issue_stage(0, 0)

        @pl.loop(0, _NCH)
        def _chunk(ci_):
            par = lax.rem(ci_, 2)
            t0 = tau0 + ci_ * _CT
            for k in range(4):
                pltpu.make_async_copy(s_hbm.at[pl.ds(0, _CT), :],
                                      ssv.at[par, k], sem_in).wait()
                pltpu.make_async_copy(i_hbm.at[pl.ds(0, _CT), :],
                                      ssi.at[par, k], sem_in).wait()

            @pl.when(ci_ + 1 < _NCH)
            def _():
                issue_stage(ci_ + 1, 1 - par)

            @pl.loop(0, _CT // 2)
            def _pair(tp):
                rows = ((2 * tp, 0), (2 * tp + 1, 1))
                atts = [[None, None], [None, None]]
                for tl, bufi in rows:
                    for p in range(2):
                        tv0 = ssv[par, p * 2 + 0, tl, :]
                        ti0 = ssi[par, p * 2 + 0, tl, :]
                        tv1 = ssv[par, p * 2 + 1, tl, :]
                        ti1 = ssi[par, p * 2 + 1, tl, :]
                        rv, ri = _combine(tv0, ti0, tv1, ti1)
                        mx = jnp.max(rv)
                        ev = jnp.exp(rv - mx)
                        atts[bufi][p] = ev / jnp.sum(ev)
                        vib[bufi, pl.ds(p * 16, 16)] = ri + vbase
                    pltpu.async_copy(values_hbm.at[vib.at[bufi]],
                                     gb.at[bufi], sem_g[bufi])
                for tl, bufi in rows:
                    pltpu.make_async_copy(values_hbm.at[vib.at[bufi]],
                                          gb.at[bufi], sem_g[bufi]).wait()
                    maskhi = jnp.int32(-65536)
                    half = jnp.int32(0x8000)
                    for p in range(2):
                        at = atts[bufi][p]
                        acca = [None] * 8
                        accb = [None] * 8
                        for j in range(16):
                            aj = at[j]
                            for dc in range(8):
                                gw = gb[bufi, p * 16 + j, pl.ds(dc * 16, 16)]
                                gi = plsc.bitcast(gw, jnp.int32)
                                ga = plsc.bitcast(gi << 16, jnp.float32)
                                gc = plsc.bitcast(gi & maskhi, jnp.float32)
                                ta = aj * ga
                                tb = aj * gc
                                acca[dc] = ta if j == 0 else acca[dc] + ta
                                accb[dc] = tb if j == 0 else accb[dc] + tb
                        for dc in range(8):
                            ia = plsc.bitcast(acca[dc], jnp.int32)
                            ib2 = plsc.bitcast(accb[dc], jnp.int32)
                            lo = lax.shift_right_logical(ia + half, 16)
                            hi = (ib2 + half) & maskhi
                            outb[2 * tl + p, pl.ds(dc * 16, 16)] = plsc.bitcast(
                                hi | lo, jnp.float32)

            pltpu.sync_copy(outb, out_hbm.at[pl.ds(out_base + 2 * t0, 2 * _CT), :])

    return body(s_flat, i_flat, values_flat)


def kernel(x, W_q, W_o, b_o, keys, values):
    b, t, e = x.shape
    wq_r = W_q.reshape(2 * _HEADS, _DH, e)
    k_r = jnp.transpose(keys, (2, 0, 1, 3)).reshape(2 * _HEADS, _NK, _DH)
    sc_top, idx_top = _proj_score(x, wq_r, k_r)
    s_flat = sc_top.reshape(b * 2 * _HEADS * t, _TK)
    i_flat = idx_top.reshape(b * 2 * _HEADS * t, _TK)
    vi = lax.bitcast_convert_type(values.reshape(_HEADS * _NK * _NK, _DH),
                                  jnp.int32) + jnp.int32(0x8000)
    vw = (vi[:, _DH // 2:] & jnp.int32(-65536)) | lax.shift_right_logical(
        vi[:, :_DH // 2], 16)
    values_flat = lax.bitcast_convert_type(vw, jnp.float32)
    out_words = _pkm_sc(s_flat, i_flat, values_flat,
                        b * _HEADS * t)
    out_words = out_words.reshape(b, _HEADS, t, _DH // 2)
    y = _out_proj(out_words, W_o, b_o.reshape(1, _DIM))
    return y


# speedup vs baseline: 6.2656x; 1.1601x over previous
import functools

import numpy as np

import jax
import jax.numpy as jnp
from jax import lax
from jax.experimental import pallas as pl
from jax.experimental.pallas import tpu as pltpu
from jax.experimental.pallas import tpu_sc as plsc

_DIM = 2048
_HEADS = 8
_NK = 128
_TK = 16
_DH = _DIM // _HEADS

_TBLK = 512


def _proj_score_body(x_ref, wq_ref, k_ref, s_ref, i_ref):
    xb = x_ref[0]
    wq = wq_ref[0]
    kk = k_ref[0]
    q = lax.dot_general(xb, wq, (((1,), (1,)), ((), ())),
                        preferred_element_type=jnp.float32)
    d_t = lax.dot_general(kk, q, (((1,), (1,)), ((), ())),
                          preferred_element_type=jnp.float32)
    work = d_t
    iot = lax.broadcasted_iota(jnp.int32, (_NK, _TBLK), 0)
    svals, sidx = [], []
    for _k in range(_TK):
        m = jnp.max(work, axis=0)
        eq = work == m[None, :]
        mi = jnp.min(jnp.where(eq, iot, _NK), axis=0)
        svals.append(m)
        sidx.append(mi)
        work = jnp.where(iot == mi[None, :], jnp.float32(-3.0e38), work)
    s_ref[0, 0] = jnp.stack(svals, axis=0).T
    i_ref[0, 0] = jnp.stack(sidx, axis=0).T


def _proj_score(x, wq_r, k_r):
    b, t, e = x.shape
    grid = (b, t // _TBLK, 2 * _HEADS)
    return pl.pallas_call(
        _proj_score_body,
        grid=grid,
        in_specs=[
            pl.BlockSpec((1, _TBLK, e), lambda ib, it, iph: (ib, it, 0)),
            pl.BlockSpec((1, _DH, e), lambda ib, it, iph: (iph, 0, 0)),
            pl.BlockSpec((1, _NK, _DH), lambda ib, it, iph: (iph, 0, 0)),
        ],
        out_specs=[
            pl.BlockSpec((1, 1, _TBLK, _TK), lambda ib, it, iph: (ib, iph, it, 0)),
            pl.BlockSpec((1, 1, _TBLK, _TK), lambda ib, it, iph: (ib, iph, it, 0)),
        ],
        out_shape=[
            jax.ShapeDtypeStruct((b, 2 * _HEADS, t, _TK), jnp.float32),
            jax.ShapeDtypeStruct((b, 2 * _HEADS, t, _TK), jnp.int32),
        ],
    )(x, wq_r, k_r)


_TBLK2 = 512


def _out_proj_body(xh_ref, we_ref, wo_ref, bo_ref, y_ref):
    ih = pl.program_id(2)
    gi = lax.bitcast_convert_type(xh_ref[0, 0], jnp.int32)
    ge = lax.bitcast_convert_type(gi << 16, jnp.float32)
    go = lax.bitcast_convert_type(gi & jnp.int32(-65536), jnp.float32)
    part = lax.dot_general(ge, we_ref[...], (((1,), (1,)), ((), ())),
                           preferred_element_type=jnp.float32)
    part += lax.dot_general(go, wo_ref[...], (((1,), (1,)), ((), ())),
                            preferred_element_type=jnp.float32)

    @pl.when(ih == 0)
    def _():
        y_ref[0] = part + bo_ref[0]

    @pl.when(ih != 0)
    def _():
        y_ref[0] += part


def _out_proj(out_words, wo, bo_r):
    b = out_words.shape[0]
    t = out_words.shape[2]
    grid = (b, t // _TBLK2, _HEADS)
    hw = _DH // 2
    return pl.pallas_call(
        _out_proj_body,
        grid=grid,
        in_specs=[
            pl.BlockSpec((1, 1, _TBLK2, hw), lambda ib, it, ih: (ib, ih, it, 0)),
            pl.BlockSpec((_DIM, hw), lambda ib, it, ih: (0, 2 * ih)),
            pl.BlockSpec((_DIM, hw), lambda ib, it, ih: (0, 2 * ih + 1)),
            pl.BlockSpec((1, _DIM), lambda ib, it, ih: (0, 0)),
        ],
        out_specs=pl.BlockSpec((1, _TBLK2, _DIM), lambda ib, it, ih: (ib, it, 0)),
        out_shape=jax.ShapeDtypeStruct((b, t, _DIM), jnp.float32),
    )(out_words, wo, wo, bo_r)


_CT = 16
_NCH = 512 // _CT

def _sortd(k, v):
    return plsc.sort_key_val(k, v, descending=True)


def _tophalf(tv, ti, cv, ci):
    rcv = jnp.flip(cv, 0)
    rci = jnp.flip(ci, 0)
    m = tv >= rcv
    return jnp.where(m, tv, rcv), jnp.where(m, ti, rci)


def _merge16(tv, ti, cv, ci):
    nv, ni = _tophalf(tv, ti, cv, ci)
    return _sortd(nv, ni)


def _combine(tv0, ti0, tv1, ti1):
    rows = [(tv1 + tv0[i], ti1 + ti0[i] * _NK) for i in range(4)]
    colmask = lax.iota(jnp.int32, 16) >= 4
    cols = []
    for j in range(3):
        cv = tv0 + tv1[j]
        ci = ti0 * _NK + ti1[j]
        sv, si, om = plsc.sort_key_val(cv, ci, mask=colmask, descending=True)
        cols.append((jnp.where(om, sv, jnp.float32(-3.0e38)), si))
    mr = _merge16(*_merge16(*rows[0], *rows[1]), *_merge16(*rows[2], *rows[3]))
    mc = _merge16(*_merge16(*cols[0], *cols[1]), *cols[2])
    return _tophalf(*mr, *mc)


def _pkm_sc(s_flat, i_flat, values_flat, n_rows_out, nb):
    per_b = 32 // nb
    nh = per_b // _HEADS
    tau_span = 1024 // nh
    nch = tau_span // _CT
    mesh = plsc.VectorSubcoreMesh(core_axis_name="c", subcore_axis_name="s",
                                  num_cores=2, num_subcores=16)

    @functools.partial(
        pl.kernel,
        out_type=jax.ShapeDtypeStruct((n_rows_out, _DH // 2), jnp.float32),
        mesh=mesh,
        compiler_params=pltpu.CompilerParams(needs_layout_passes=False),
        scratch_types=[
            pltpu.VMEM((2, 4, _CT, _TK), jnp.float32),
            pltpu.VMEM((2, 4, _CT, _TK), jnp.int32),
            pltpu.VMEM((2 * _CT, _DH // 2), jnp.float32),
            pltpu.VMEM((2, 32, _DH // 2), jnp.float32),
            pltpu.VMEM((2, 32), jnp.int32),
            pltpu.SemaphoreType.DMA,
            pltpu.SemaphoreType.DMA,
            pltpu.SemaphoreType.DMA,
        ],
    )
    def body(s_hbm, i_hbm, values_hbm, out_hbm,
             ssv, ssi, outb, gb, vib, sem_in, sem_g0, sem_g1):
        c = lax.axis_index("c")
        s = lax.axis_index("s")
        w = c * 16 + s
        wb = w // per_b
        r = lax.rem(w, per_b)
        h = r // nh
        hq = lax.rem(r, nh)
        row_p = [(wb * 16 + h) * 2048, (wb * 16 + 8 + h) * 2048]
        tau0 = hq * tau_span
        out_base = (wb * 8 + h) * 2048
        vbase = h * (_NK * _NK)
        sem_g = [sem_g0, sem_g1]

        def issue_stage(ci_, par):
            t0 = tau0 + ci_ * _CT
            for p in range(2):
                for seg in range(2):
                    rb = row_p[p] + seg * 1024 + t0
                    pltpu.async_copy(s_hbm.at[pl.ds(rb, _CT), :],
                                     ssv.at[par, p * 2 + seg], sem_in)
                    pltpu.async_copy(i_hbm.at[pl.ds(rb, _CT), :],
                                     ssi.at[par, p * 2 + seg], sem_in)

        issue_stage(0, 0)

        @pl.loop(0, nch)
        def _chunk(ci_):
            par = lax.rem(ci_, 2)
            t0 = tau0 + ci_ * _CT
            for k in range(4):
                pltpu.make_async_copy(s_hbm.at[pl.ds(0, _CT), :],
                                      ssv.at[par, k], sem_in).wait()
                pltpu.make_async_copy(i_hbm.at[pl.ds(0, _CT), :],
                                      ssi.at[par, k], sem_in).wait()

            @pl.when(ci_ + 1 < nch)
            def _():
                issue_stage(ci_ + 1, 1 - par)

            @pl.loop(0, _CT // 2)
            def _pair(tp):
                rows = ((2 * tp, 0), (2 * tp + 1, 1))
                atts = [[None, None], [None, None]]
                for tl, bufi in rows:
                    for p in range(2):
                        tv0 = ssv[par, p * 2 + 0, tl, :]
                        ti0 = ssi[par, p * 2 + 0, tl, :]
                        tv1 = ssv[par, p * 2 + 1, tl, :]
                        ti1 = ssi[par, p * 2 + 1, tl, :]
                        rv, ri = _combine(tv0, ti0, tv1, ti1)
                        mx = jnp.max(rv)
                        ev = jnp.exp(rv - mx)
                        atts[bufi][p] = ev / jnp.sum(ev)
                        vib[bufi, pl.ds(p * 16, 16)] = ri + vbase
                    pltpu.async_copy(values_hbm.at[vib.at[bufi]],
                                     gb.at[bufi], sem_g[bufi])
                for tl, bufi in rows:
                    pltpu.make_async_copy(values_hbm.at[vib.at[bufi]],
                                          gb.at[bufi], sem_g[bufi]).wait()
                    maskhi = jnp.int32(-65536)
                    half = jnp.int32(0x8000)
                    for p in range(2):
                        at = atts[bufi][p]
                        acca = [None] * 8
                        accb = [None] * 8
                        for j in range(16):
                            aj = at[j]
                            for dc in range(8):
                                gw = gb[bufi, p * 16 + j, pl.ds(dc * 16, 16)]
                                gi = plsc.bitcast(gw, jnp.int32)
                                ga = plsc.bitcast(gi << 16, jnp.float32)
                                gc = plsc.bitcast(gi & maskhi, jnp.float32)
                                ta = aj * ga
                                tb = aj * gc
                                acca[dc] = ta if j == 0 else acca[dc] + ta
                                accb[dc] = tb if j == 0 else accb[dc] + tb
                        for dc in range(8):
                            ia = plsc.bitcast(acca[dc], jnp.int32)
                            ib2 = plsc.bitcast(accb[dc], jnp.int32)
                            lo = lax.shift_right_logical(ia + half, 16)
                            hi = (ib2 + half) & maskhi
                            outb[2 * tl + p, pl.ds(dc * 16, 16)] = plsc.bitcast(
                                hi | lo, jnp.float32)

            pltpu.sync_copy(outb, out_hbm.at[pl.ds(out_base + 2 * t0, 2 * _CT), :])

    return body(s_flat, i_flat, values_flat)


def kernel(x, W_q, W_o, b_o, keys, values):
    b, t, e = x.shape
    wq_r = W_q.reshape(2 * _HEADS, _DH, e)
    k_r = jnp.transpose(keys, (2, 0, 1, 3)).reshape(2 * _HEADS, _NK, _DH)
    def stage_a(xh):
        sc_top, idx_top = _proj_score(xh, wq_r, k_r)
        return (sc_top.reshape(2 * _HEADS * t, _TK),
                idx_top.reshape(2 * _HEADS * t, _TK))
    vi = lax.bitcast_convert_type(values.reshape(_HEADS * _NK * _NK, _DH),
                                  jnp.int32) + jnp.int32(0x8000)
    vw = (vi[:, _DH // 2:] & jnp.int32(-65536)) | lax.shift_right_logical(
        vi[:, :_DH // 2], 16)
    values_flat = lax.bitcast_convert_type(vw, jnp.float32)
    bo_r = b_o.reshape(1, _DIM)
    ys = []
    flats = [stage_a(x[ib:ib + 1]) for ib in range(b)]
    for ib in range(b):
        s_flat, i_flat = flats[ib]
        ow = _pkm_sc(s_flat, i_flat, values_flat, _HEADS * t, nb=1)
        ow = ow.reshape(1, _HEADS, t, _DH // 2)
        ys.append(_out_proj(ow, W_o, bo_r))
    return jnp.concatenate(ys, axis=0)


# speedup vs baseline: 6.4162x; 1.0240x over previous
import functools

import numpy as np

import jax
import jax.numpy as jnp
from jax import lax
from jax.experimental import pallas as pl
from jax.experimental.pallas import tpu as pltpu
from jax.experimental.pallas import tpu_sc as plsc

_DIM = 2048
_HEADS = 8
_NK = 128
_TK = 16
_DH = _DIM // _HEADS

_TBLK = 512


def _proj_score_body(x_ref, wq_ref, k_ref, s_ref, i_ref):
    xb = x_ref[0]
    wq = wq_ref[0]
    kk = k_ref[0]
    q = lax.dot_general(xb, wq, (((1,), (1,)), ((), ())),
                        preferred_element_type=jnp.float32)
    d_t = lax.dot_general(kk, q, (((1,), (1,)), ((), ())),
                          preferred_element_type=jnp.float32)
    work = d_t
    iot = lax.broadcasted_iota(jnp.int32, (_NK, _TBLK), 0)
    svals, sidx = [], []
    for _k in range(_TK):
        m = jnp.max(work, axis=0)
        eq = work == m[None, :]
        mi = jnp.min(jnp.where(eq, iot, _NK), axis=0)
        svals.append(m)
        sidx.append(mi)
        work = jnp.where(iot == mi[None, :], jnp.float32(-3.0e38), work)
    s_ref[0, 0] = jnp.stack(svals, axis=0).T
    i_ref[0, 0] = jnp.stack(sidx, axis=0).T


def _proj_score(x, wq_r, k_r):
    b, t, e = x.shape
    grid = (b, t // _TBLK, 2 * _HEADS)
    return pl.pallas_call(
        _proj_score_body,
        grid=grid,
        in_specs=[
            pl.BlockSpec((1, _TBLK, e), lambda ib, it, iph: (ib, it, 0)),
            pl.BlockSpec((1, _DH, e), lambda ib, it, iph: (iph, 0, 0)),
            pl.BlockSpec((1, _NK, _DH), lambda ib, it, iph: (iph, 0, 0)),
        ],
        out_specs=[
            pl.BlockSpec((1, 1, _TBLK, _TK), lambda ib, it, iph: (ib, iph, it, 0)),
            pl.BlockSpec((1, 1, _TBLK, _TK), lambda ib, it, iph: (ib, iph, it, 0)),
        ],
        out_shape=[
            jax.ShapeDtypeStruct((b, 2 * _HEADS, t, _TK), jnp.float32),
            jax.ShapeDtypeStruct((b, 2 * _HEADS, t, _TK), jnp.int32),
        ],
    )(x, wq_r, k_r)


_TBLK2 = 512


def _out_proj_body(xh_ref, we_ref, wo_ref, bo_ref, y_ref):
    ih = pl.program_id(2)
    gi = lax.bitcast_convert_type(xh_ref[0, 0], jnp.int32)
    ge = lax.bitcast_convert_type(gi << 16, jnp.float32)
    go = lax.bitcast_convert_type(gi & jnp.int32(-65536), jnp.float32)
    part = lax.dot_general(ge, we_ref[...], (((1,), (1,)), ((), ())),
                           preferred_element_type=jnp.float32)
    part += lax.dot_general(go, wo_ref[...], (((1,), (1,)), ((), ())),
                            preferred_element_type=jnp.float32)

    @pl.when(ih == 0)
    def _():
        y_ref[0] = part + bo_ref[0]

    @pl.when(ih != 0)
    def _():
        y_ref[0] += part


def _out_proj(out_words, wo, bo_r):
    b = out_words.shape[0]
    t = out_words.shape[2]
    grid = (b, t // _TBLK2, _HEADS)
    hw = _DH // 2
    return pl.pallas_call(
        _out_proj_body,
        grid=grid,
        in_specs=[
            pl.BlockSpec((1, 1, _TBLK2, hw), lambda ib, it, ih: (ib, ih, it, 0)),
            pl.BlockSpec((_DIM, hw), lambda ib, it, ih: (0, 2 * ih)),
            pl.BlockSpec((_DIM, hw), lambda ib, it, ih: (0, 2 * ih + 1)),
            pl.BlockSpec((1, _DIM), lambda ib, it, ih: (0, 0)),
        ],
        out_specs=pl.BlockSpec((1, _TBLK2, _DIM), lambda ib, it, ih: (ib, it, 0)),
        out_shape=jax.ShapeDtypeStruct((b, t, _DIM), jnp.float32),
    )(out_words, wo, wo, bo_r)


_CT = 16
_NCH = 512 // _CT

def _sortd(k, v):
    return plsc.sort_key_val(k, v, descending=True)


def _tophalf(tv, ti, cv, ci):
    rcv = jnp.flip(cv, 0)
    rci = jnp.flip(ci, 0)
    m = tv >= rcv
    return jnp.where(m, tv, rcv), jnp.where(m, ti, rci)


def _merge16(tv, ti, cv, ci):
    nv, ni = _tophalf(tv, ti, cv, ci)
    return _sortd(nv, ni)


def _combine(tv0, ti0, tv1, ti1):
    rows = [(tv1 + tv0[i], ti1 + ti0[i] * _NK) for i in range(4)]
    colmask = lax.iota(jnp.int32, 16) >= 4
    cols = []
    for j in range(3):
        cv = tv0 + tv1[j]
        ci = ti0 * _NK + ti1[j]
        sv, si, om = plsc.sort_key_val(cv, ci, mask=colmask, descending=True)
        cols.append((jnp.where(om, sv, jnp.float32(-3.0e38)), si))
    mr = _merge16(*_merge16(*rows[0], *rows[1]), *_merge16(*rows[2], *rows[3]))
    mc = _merge16(*_merge16(*cols[0], *cols[1]), *cols[2])
    return _tophalf(*mr, *mc)


def _pkm_sc(s_flat, i_flat, values_flat, n_rows_out, nb):
    per_b = 32 // nb
    nh = per_b // _HEADS
    tau_span = 1024 // nh
    nch = tau_span // _CT
    mesh = plsc.VectorSubcoreMesh(core_axis_name="c", subcore_axis_name="s",
                                  num_cores=2, num_subcores=16)

    @functools.partial(
        pl.kernel,
        out_type=jax.ShapeDtypeStruct((n_rows_out, _DH // 2), jnp.float32),
        mesh=mesh,
        compiler_params=pltpu.CompilerParams(needs_layout_passes=False),
        scratch_types=[
            pltpu.VMEM((2, 4, _CT, _TK), jnp.float32),
            pltpu.VMEM((2, 4, _CT, _TK), jnp.int32),
            pltpu.VMEM((2 * _CT, _DH // 2), jnp.float32),
            pltpu.VMEM((2, 32, _DH // 2), jnp.float32),
            pltpu.VMEM((2, 32), jnp.int32),
            pltpu.SemaphoreType.DMA,
            pltpu.SemaphoreType.DMA,
            pltpu.SemaphoreType.DMA,
        ],
    )
    def body(s_hbm, i_hbm, values_hbm, out_hbm,
             ssv, ssi, outb, gb, vib, sem_in, sem_g0, sem_g1):
        c = lax.axis_index("c")
        s = lax.axis_index("s")
        w = c * 16 + s
        wb = w // per_b
        r = lax.rem(w, per_b)
        h = r // nh
        hq = lax.rem(r, nh)
        row_p = [(wb * 16 + h) * 2048, (wb * 16 + 8 + h) * 2048]
        tau0 = hq * tau_span
        out_base = (wb * 8 + h) * 2048
        vbase = h * (_NK * _NK)
        sem_g = [sem_g0, sem_g1]

        def issue_stage(ci_, par):
            t0 = tau0 + ci_ * _CT
            for p in range(2):
                for seg in range(2):
                    rb = row_p[p] + seg * 1024 + t0
                    pltpu.async_copy(s_hbm.at[pl.ds(rb, _CT), :],
                                     ssv.at[par, p * 2 + seg], sem_in)
                    pltpu.async_copy(i_hbm.at[pl.ds(rb, _CT), :],
                                     ssi.at[par, p * 2 + seg], sem_in)

        issue_stage(0, 0)

        @pl.loop(0, nch)
        def _chunk(ci_):
            par = lax.rem(ci_, 2)
            t0 = tau0 + ci_ * _CT
            for k in range(4):
                pltpu.make_async_copy(s_hbm.at[pl.ds(0, _CT), :],
                                      ssv.at[par, k], sem_in).wait()
                pltpu.make_async_copy(i_hbm.at[pl.ds(0, _CT), :],
                                      ssi.at[par, k], sem_in).wait()

            @pl.when(ci_ + 1 < nch)
            def _():
                issue_stage(ci_ + 1, 1 - par)

            @pl.loop(0, _CT // 2)
            def _pair(tp):
                rows = ((2 * tp, 0), (2 * tp + 1, 1))
                atts = [[None, None], [None, None]]
                for tl, bufi in rows:
                    for p in range(2):
                        tv0 = ssv[par, p * 2 + 0, tl, :]
                        ti0 = ssi[par, p * 2 + 0, tl, :]
                        tv1 = ssv[par, p * 2 + 1, tl, :]
                        ti1 = ssi[par, p * 2 + 1, tl, :]
                        rv, ri = _combine(tv0, ti0, tv1, ti1)
                        mx = jnp.max(rv)
                        ev = jnp.exp(rv - mx)
                        atts[bufi][p] = ev / jnp.sum(ev)
                        vib[bufi, pl.ds(p * 16, 16)] = ri + vbase
                    pltpu.async_copy(values_hbm.at[vib.at[bufi]],
                                     gb.at[bufi], sem_g[bufi])
                for tl, bufi in rows:
                    pltpu.make_async_copy(values_hbm.at[vib.at[bufi]],
                                          gb.at[bufi], sem_g[bufi]).wait()
                    maskhi = jnp.int32(-65536)
                    half = jnp.int32(0x8000)
                    for p in range(2):
                        at = atts[bufi][p]
                        acca = [None] * 8
                        accb = [None] * 8
                        for j in range(16):
                            aj = at[j]
                            for dc in range(8):
                                gw = gb[bufi, p * 16 + j, pl.ds(dc * 16, 16)]
                                gi = plsc.bitcast(gw, jnp.int32)
                                ga = plsc.bitcast(gi << 16, jnp.float32)
                                gc = plsc.bitcast(gi, jnp.float32)
                                ta = aj * ga
                                tb = aj * gc
                                acca[dc] = ta if j == 0 else acca[dc] + ta
                                accb[dc] = tb if j == 0 else accb[dc] + tb
                        for dc in range(8):
                            ia = plsc.bitcast(acca[dc], jnp.int32)
                            ib2 = plsc.bitcast(accb[dc], jnp.int32)
                            lo = lax.shift_right_logical(ia + half, 16)
                            hi = (ib2 + half) & maskhi
                            outb[2 * tl + p, pl.ds(dc * 16, 16)] = plsc.bitcast(
                                hi | lo, jnp.float32)

            pltpu.sync_copy(outb, out_hbm.at[pl.ds(out_base + 2 * t0, 2 * _CT), :])

    return body(s_flat, i_flat, values_flat)


def kernel(x, W_q, W_o, b_o, keys, values):
    b, t, e = x.shape
    wq_r = W_q.reshape(2 * _HEADS, _DH, e)
    k_r = jnp.transpose(keys, (2, 0, 1, 3)).reshape(2 * _HEADS, _NK, _DH)
    def stage_a(xh):
        sc_top, idx_top = _proj_score(xh, wq_r, k_r)
        return (sc_top.reshape(2 * _HEADS * t, _TK),
                idx_top.reshape(2 * _HEADS * t, _TK))
    vi = lax.bitcast_convert_type(values.reshape(_HEADS * _NK * _NK, _DH),
                                  jnp.int32) + jnp.int32(0x8000)
    vw = (vi[:, _DH // 2:] & jnp.int32(-65536)) | lax.shift_right_logical(
        vi[:, :_DH // 2], 16)
    values_flat = lax.bitcast_convert_type(vw, jnp.float32)
    bo_r = b_o.reshape(1, _DIM)
    ys = []
    flats = [stage_a(x[ib:ib + 1]) for ib in range(b)]
    for ib in range(b):
        s_flat, i_flat = flats[ib]
        ow = _pkm_sc(s_flat, i_flat, values_flat, _HEADS * t, nb=1)
        ow = ow.reshape(1, _HEADS, t, _DH // 2)
        ys.append(_out_proj(ow, W_o, bo_r))
    return jnp.concatenate(ys, axis=0)


# speedup vs baseline: 6.4220x; 1.0009x over previous
import functools

import jax
import jax.numpy as jnp
from jax import lax
from jax.experimental import pallas as pl
from jax.experimental.pallas import tpu as pltpu
from jax.experimental.pallas import tpu_sc as plsc

_DIM = 2048
_HEADS = 8
_NK = 128
_TK = 16
_DH = _DIM // _HEADS

_TBLK = 512


def _proj_score_body(x_ref, wq_ref, k_ref, s_ref, i_ref):
    xb = x_ref[0]
    wq = wq_ref[0]
    kk = k_ref[0]
    q = lax.dot_general(xb, wq, (((1,), (1,)), ((), ())),
                        preferred_element_type=jnp.float32)
    d_t = lax.dot_general(kk, q, (((1,), (1,)), ((), ())),
                          preferred_element_type=jnp.float32)
    work = d_t
    iot = lax.broadcasted_iota(jnp.int32, (_NK, _TBLK), 0)
    svals, sidx = [], []
    for _k in range(_TK):
        m = jnp.max(work, axis=0)
        eq = work == m[None, :]
        mi = jnp.min(jnp.where(eq, iot, _NK), axis=0)
        svals.append(m)
        sidx.append(mi)
        work = jnp.where(iot == mi[None, :], jnp.float32(-3.0e38), work)
    s_ref[0, 0] = jnp.stack(svals, axis=0).T
    i_ref[0, 0] = jnp.stack(sidx, axis=0).T


def _proj_score(x, wq_r, k_r):
    b, t, e = x.shape
    grid = (b, t // _TBLK, 2 * _HEADS)
    return pl.pallas_call(
        _proj_score_body,
        grid=grid,
        in_specs=[
            pl.BlockSpec((1, _TBLK, e), lambda ib, it, iph: (ib, it, 0)),
            pl.BlockSpec((1, _DH, e), lambda ib, it, iph: (iph, 0, 0)),
            pl.BlockSpec((1, _NK, _DH), lambda ib, it, iph: (iph, 0, 0)),
        ],
        out_specs=[
            pl.BlockSpec((1, 1, _TBLK, _TK), lambda ib, it, iph: (ib, iph, it, 0)),
            pl.BlockSpec((1, 1, _TBLK, _TK), lambda ib, it, iph: (ib, iph, it, 0)),
        ],
        out_shape=[
            jax.ShapeDtypeStruct((b, 2 * _HEADS, t, _TK), jnp.float32),
            jax.ShapeDtypeStruct((b, 2 * _HEADS, t, _TK), jnp.int32),
        ],
    )(x, wq_r, k_r)


_TBLK2 = 512


def _out_proj_body(xh_ref, we_ref, wo_ref, bo_ref, y_ref):
    ih = pl.program_id(2)
    gi = lax.bitcast_convert_type(xh_ref[0, 0], jnp.int32)
    ge = lax.bitcast_convert_type(gi << 16, jnp.float32)
    go = lax.bitcast_convert_type(gi & jnp.int32(-65536), jnp.float32)
    part = lax.dot_general(ge, we_ref[...], (((1,), (1,)), ((), ())),
                           preferred_element_type=jnp.float32)
    part += lax.dot_general(go, wo_ref[...], (((1,), (1,)), ((), ())),
                            preferred_element_type=jnp.float32)

    @pl.when(ih == 0)
    def _():
        y_ref[0] = part + bo_ref[0]

    @pl.when(ih != 0)
    def _():
        y_ref[0] += part


def _out_proj(out_words, wo, bo_r):
    b = out_words.shape[0]
    t = out_words.shape[2]
    grid = (b, t // _TBLK2, _HEADS)
    hw = _DH // 2
    return pl.pallas_call(
        _out_proj_body,
        grid=grid,
        in_specs=[
            pl.BlockSpec((1, 1, _TBLK2, hw), lambda ib, it, ih: (ib, ih, it, 0)),
            pl.BlockSpec((_DIM, hw), lambda ib, it, ih: (0, 2 * ih)),
            pl.BlockSpec((_DIM, hw), lambda ib, it, ih: (0, 2 * ih + 1)),
            pl.BlockSpec((1, _DIM), lambda ib, it, ih: (0, 0)),
        ],
        out_specs=pl.BlockSpec((1, _TBLK2, _DIM), lambda ib, it, ih: (ib, it, 0)),
        out_shape=jax.ShapeDtypeStruct((b, t, _DIM), jnp.float32),
    )(out_words, wo, wo, bo_r)


_CT = 16

def _sortd(k, v):
    return plsc.sort_key_val(k, v, descending=True)


def _tophalf(tv, ti, cv, ci):
    rcv = jnp.flip(cv, 0)
    rci = jnp.flip(ci, 0)
    m = tv >= rcv
    return jnp.where(m, tv, rcv), jnp.where(m, ti, rci)


def _merge16(tv, ti, cv, ci):
    nv, ni = _tophalf(tv, ti, cv, ci)
    return _sortd(nv, ni)


def _combine(tv0, ti0, tv1, ti1):
    rows = [(tv1 + tv0[i], ti1 + ti0[i] * _NK) for i in range(4)]
    colmask = lax.iota(jnp.int32, 16) >= 4
    cols = []
    for j in range(3):
        cv = tv0 + tv1[j]
        ci = ti0 * _NK + ti1[j]
        sv, si, om = plsc.sort_key_val(cv, ci, mask=colmask, descending=True)
        cols.append((jnp.where(om, sv, jnp.float32(-3.0e38)), si))
    mr = _merge16(*_merge16(*rows[0], *rows[1]), *_merge16(*rows[2], *rows[3]))
    mc = _merge16(*_merge16(*cols[0], *cols[1]), *cols[2])
    return _tophalf(*mr, *mc)


def _pkm_sc(s_flat, i_flat, values_flat, n_rows_out, nb):
    per_b = 32 // nb
    nh = per_b // _HEADS
    tau_span = 1024 // nh
    nch = tau_span // _CT
    mesh = plsc.VectorSubcoreMesh(core_axis_name="c", subcore_axis_name="s",
                                  num_cores=2, num_subcores=16)

    @functools.partial(
        pl.kernel,
        out_type=jax.ShapeDtypeStruct((n_rows_out, _DH // 2), jnp.float32),
        mesh=mesh,
        compiler_params=pltpu.CompilerParams(needs_layout_passes=False),
        scratch_types=[
            pltpu.VMEM((2, 4, _CT, _TK), jnp.float32),
            pltpu.VMEM((2, 4, _CT, _TK), jnp.int32),
            pltpu.VMEM((2 * _CT, _DH // 2), jnp.float32),
            pltpu.VMEM((2, 32, _DH // 2), jnp.float32),
            pltpu.VMEM((2, 32), jnp.int32),
            pltpu.SemaphoreType.DMA,
            pltpu.SemaphoreType.DMA,
            pltpu.SemaphoreType.DMA,
        ],
    )
    def body(s_hbm, i_hbm, values_hbm, out_hbm,
             ssv, ssi, outb, gb, vib, sem_in, sem_g0, sem_g1):
        c = lax.axis_index("c")
        s = lax.axis_index("s")
        w = c * 16 + s
        wb = w // per_b
        r = lax.rem(w, per_b)
        h = r // nh
        hq = lax.rem(r, nh)
        row_p = [(wb * 16 + h) * 2048, (wb * 16 + 8 + h) * 2048]
        tau0 = hq * tau_span
        out_base = (wb * 8 + h) * 2048
        vbase = h * (_NK * _NK)
        sem_g = [sem_g0, sem_g1]

        def issue_stage(ci_, par):
            t0 = tau0 + ci_ * _CT
            for p in range(2):
                for seg in range(2):
                    rb = row_p[p] + seg * 1024 + t0
                    pltpu.async_copy(s_hbm.at[pl.ds(rb, _CT), :],
                                     ssv.at[par, p * 2 + seg], sem_in)
                    pltpu.async_copy(i_hbm.at[pl.ds(rb, _CT), :],
                                     ssi.at[par, p * 2 + seg], sem_in)

        issue_stage(0, 0)

        @pl.loop(0, nch)
        def _chunk(ci_):
            par = lax.rem(ci_, 2)
            t0 = tau0 + ci_ * _CT
            for k in range(4):
                pltpu.make_async_copy(s_hbm.at[pl.ds(0, _CT), :],
                                      ssv.at[par, k], sem_in).wait()
                pltpu.make_async_copy(i_hbm.at[pl.ds(0, _CT), :],
                                      ssi.at[par, k], sem_in).wait()

            @pl.when(ci_ + 1 < nch)
            def _():
                issue_stage(ci_ + 1, 1 - par)

            @pl.loop(0, _CT // 2)
            def _pair(tp):
                rows = ((2 * tp, 0), (2 * tp + 1, 1))
                atts = [[None, None], [None, None]]
                for tl, bufi in rows:
                    for p in range(2):
                        tv0 = ssv[par, p * 2 + 0, tl, :]
                        ti0 = ssi[par, p * 2 + 0, tl, :]
                        tv1 = ssv[par, p * 2 + 1, tl, :]
                        ti1 = ssi[par, p * 2 + 1, tl, :]
                        rv, ri = _combine(tv0, ti0, tv1, ti1)
                        mx = jnp.max(rv)
                        ev = jnp.exp(rv - mx)
                        atts[bufi][p] = ev / jnp.sum(ev)
                        vib[bufi, pl.ds(p * 16, 16)] = ri + vbase
                    pltpu.async_copy(values_hbm.at[vib.at[bufi]],
                                     gb.at[bufi], sem_g[bufi])
                for tl, bufi in rows:
                    pltpu.make_async_copy(values_hbm.at[vib.at[bufi]],
                                          gb.at[bufi], sem_g[bufi]).wait()
                    maskhi = jnp.int32(-65536)
                    half = jnp.int32(0x8000)
                    for p in range(2):
                        at = atts[bufi][p]
                        acca = [None] * 8
                        accb = [None] * 8
                        for j in range(16):
                            aj = at[j]
                            for dc in range(8):
                                gw = gb[bufi, p * 16 + j, pl.ds(dc * 16, 16)]
                                gi = plsc.bitcast(gw, jnp.int32)
                                ga = plsc.bitcast(gi << 16, jnp.float32)
                                gc = plsc.bitcast(gi, jnp.float32)
                                ta = aj * ga
                                tb = aj * gc
                                acca[dc] = ta if j == 0 else acca[dc] + ta
                                accb[dc] = tb if j == 0 else accb[dc] + tb
                        for dc in range(8):
                            ia = plsc.bitcast(acca[dc], jnp.int32)
                            ib2 = plsc.bitcast(accb[dc], jnp.int32)
                            lo = lax.shift_right_logical(ia + half, 16)
                            hi = (ib2 + half) & maskhi
                            outb[2 * tl + p, pl.ds(dc * 16, 16)] = plsc.bitcast(
                                hi | lo, jnp.float32)

            pltpu.sync_copy(outb, out_hbm.at[pl.ds(out_base + 2 * t0, 2 * _CT), :])

    return body(s_flat, i_flat, values_flat)


def kernel(x, W_q, W_o, b_o, keys, values):
    b, t, e = x.shape
    wq_r = W_q.reshape(2 * _HEADS, _DH, e)
    k_r = jnp.transpose(keys, (2, 0, 1, 3)).reshape(2 * _HEADS, _NK, _DH)
    def stage_a(xh):
        sc_top, idx_top = _proj_score(xh, wq_r, k_r)
        return (sc_top.reshape(2 * _HEADS * t, _TK),
                idx_top.reshape(2 * _HEADS * t, _TK))
    vi = lax.bitcast_convert_type(values.reshape(_HEADS * _NK * _NK, _DH),
                                  jnp.int32) + jnp.int32(0x8000)
    vw = (vi[:, _DH // 2:] & jnp.int32(-65536)) | lax.shift_right_logical(
        vi[:, :_DH // 2], 16)
    values_flat = lax.bitcast_convert_type(vw, jnp.float32)
    bo_r = b_o.reshape(1, _DIM)
    ys = []
    flats = [stage_a(x[ib:ib + 1]) for ib in range(b)]
    for ib in range(b):
        s_flat, i_flat = flats[ib]
        ow = _pkm_sc(s_flat, i_flat, values_flat, _HEADS * t, nb=1)
        ow = ow.reshape(1, _HEADS, t, _DH // 2)
        ys.append(_out_proj(ow, W_o, bo_r))
    return jnp.concatenate(ys, axis=0)
